# baseline (speedup 1.0000x reference)
import functools

import jax
import jax.numpy as jnp
from jax import lax
from jax.experimental import pallas as pl
from jax.experimental.pallas import tpu as pltpu
from jax.experimental.pallas import tpu_sc as plsc

N = 131072
D = 128
H = 8
G = 1024
F = 1024
E = 16384
GG = G * G

def _gates_body(ea_t_ref, egs_ref, w2_ref, b2_ref, w1t_ref, b1t_ref,
                g_ref, cw_ref, cb_ref):
    z = jnp.dot(egs_ref[...], ea_t_ref[...], preferred_element_type=jnp.float32)
    g_ref[...] = jax.nn.sigmoid(z)
    lane_blk = lax.broadcasted_iota(jnp.int32, (1, D), 1) // H
    w1t = w1t_ref[...]
    for dd in range(16):
        b3dd = jnp.where(lane_blk == dd, w1t, 0.0)
        cw_ref[dd] = jnp.dot(w2_ref[...], b3dd,
                             preferred_element_type=jnp.float32)
    cb_ref[...] = jnp.dot(b2_ref[...], w1t,
                          preferred_element_type=jnp.float32) + b1t_ref[...]


def _gates_call(ea_t, egs, w2, b2, w1t, b1t):
    return pl.pallas_call(
        _gates_body,
        out_shape=(
            jax.ShapeDtypeStruct((2, E), jnp.float32),
            jax.ShapeDtypeStruct((16, D, D), jnp.float32),
            jax.ShapeDtypeStruct((1, D), jnp.float32),
        ),
    )(ea_t, egs, w2, b2, w1t, b1t)


_NSUB = 16
_EPW = E // _NSUB
_ZB = 16384
_OUTW = GG // _NSUB
_ROWS_PER_SUB = G // _NSUB


def _adj_body(ei_hbm, gates_hbm, out_hbm, src_v, dst_v, idx_v, val_v, zero_v,
              a_sh, sem):
    cid = lax.axis_index("c")
    sid = lax.axis_index("s")

    base = sid * _EPW
    pltpu.sync_copy(ei_hbm.at[0, pl.ds(base, _EPW)], src_v)
    pltpu.sync_copy(ei_hbm.at[1, pl.ds(base, _EPW)], dst_v)
    pltpu.sync_copy(gates_hbm.at[cid, pl.ds(base, _EPW)], val_v)

    def _zfill(i, _):
        zero_v[pl.ds(i * 16, 16)] = jnp.zeros((16,), jnp.float32)
        return 0
    lax.fori_loop(0, _ZB // 16, _zfill, 0)
    for j in range(_OUTW // _ZB):
        pltpu.sync_copy(zero_v, a_sh.at[pl.ds(sid * _OUTW + j * _ZB, _ZB)])

    def _idx(i, _):
        s16 = src_v[pl.ds(i * 16, 16)]
        d16 = dst_v[pl.ds(i * 16, 16)]
        idx_v[pl.ds(i * 16, 16)] = d16 * G + s16
        return 0
    lax.fori_loop(0, _EPW // 16, _idx, 0)

    plsc.subcore_barrier()

    pltpu.sync_copy(val_v, a_sh.at[idx_v], add=True)

    plsc.subcore_barrier()

    copies = []
    for r in range(_ROWS_PER_SUB):
        row = sid * _ROWS_PER_SUB + r
        copies.append(pltpu.make_async_copy(
            a_sh.at[pl.ds(row * G, G)],
            out_hbm.at[cid * G + row],
            sem,
        ))
    for cp in copies:
        cp.start()
    for cp in copies:
        cp.wait()


_adj_call = functools.partial(
    pl.kernel,
    mesh=plsc.VectorSubcoreMesh(core_axis_name="c", subcore_axis_name="s"),
    out_type=jax.ShapeDtypeStruct((2 * G, G), jnp.float32),
    scratch_types=[
        pltpu.VMEM((_EPW,), jnp.int32),
        pltpu.VMEM((_EPW,), jnp.int32),
        pltpu.VMEM((_EPW,), jnp.int32),
        pltpu.VMEM((_EPW,), jnp.float32),
        pltpu.VMEM((_ZB,), jnp.float32),
        pltpu.VMEM_SHARED((GG,), jnp.float32),
        pltpu.SemaphoreType.DMA,
    ],
)(_adj_body)


_BN = 32768
_BR = _BN // 16


def _fc_body(x_ref, cw_ref, cb_ref, o_ref):
    xw = x_ref[...].reshape(_BR, 16 * D)
    acc = jnp.dot(xw, cw_ref[...].reshape(16 * D, D),
                  preferred_element_type=jnp.float32)
    o_ref[...] = jnp.maximum(acc + cb_ref[...], 0.0)


def _fc_call(x, cw, cb):
    return pl.pallas_call(
        _fc_body,
        grid=(N // _BN,),
        in_specs=[
            pl.BlockSpec((_BN, D), lambda i: (i, 0)),
            pl.BlockSpec((16, D, D), lambda i: (0, 0, 0)),
            pl.BlockSpec((1, D), lambda i: (0, 0)),
        ],
        out_specs=pl.BlockSpec((_BR, D), lambda i: (i, 0)),
        out_shape=jax.ShapeDtypeStruct((N // 16, D), jnp.float32),
    )(x, cw, cb)


def _convs_body(h2_ref, a_ref, wm0_ref, wr0_ref, b0_ref,
                wm1_ref, wr1_ref, b1_ref, o_ref):
    h = h2_ref[...].reshape(G, F)
    hm = jnp.dot(h, wm0_ref[...], preferred_element_type=jnp.float32)
    hr = jnp.dot(h, wr0_ref[...], preferred_element_type=jnp.float32)
    a0 = a_ref[pl.ds(0, G), :]
    a1 = a_ref[pl.ds(G, G), :]
    agg = jnp.dot(a0, hm, preferred_element_type=jnp.float32)
    h1 = jnp.maximum(hr + agg + b0_ref[...], 0.0)
    hm1 = jnp.dot(h1, wm1_ref[...], preferred_element_type=jnp.float32)
    agg1 = jnp.dot(a1, hm1, preferred_element_type=jnp.float32)
    hr1 = jnp.dot(h1, wr1_ref[...], preferred_element_type=jnp.float32)
    o_ref[...] = jnp.maximum(hr1 + agg1 + b1_ref[...], 0.0)


def _convs_call(h2, adj, wm0, wr0, b0, wm1, wr1, b1):
    return pl.pallas_call(
        _convs_body,
        out_shape=jax.ShapeDtypeStruct((G, F), jnp.float32),
    )(h2, adj, wm0, wr0, b0, wm1, wr1, b1)


def kernel(x, edge_index, edge_attr, fc2_w, fc2_b, fc1_w, fc1_b,
           conv0_root, conv0_msg, conv0_eg, conv0_b,
           conv1_root, conv1_msg, conv1_eg, conv1_b):
    ea_t = edge_attr.T
    egs = jnp.concatenate([conv0_eg.T, conv1_eg.T], axis=0)

    w1t = jnp.tile(fc1_w, (1, 16))
    b1t = jnp.tile(fc1_b, 16).reshape(1, D)

    gates, cw, cb = _gates_call(ea_t, egs, fc2_w, fc2_b.reshape(1, D),
                                w1t, b1t)

    adj = _adj_call(edge_index, gates)

    h2 = _fc_call(x, cw, cb)

    return _convs_call(h2, adj, conv0_msg, conv0_root, conv0_b.reshape(1, F),
                       conv1_msg, conv1_root, conv1_b.reshape(1, F))


# speedup vs baseline: 1.0249x; 1.0249x over previous
import functools

import jax
import jax.numpy as jnp
from jax import lax
from jax.experimental import pallas as pl
from jax.experimental.pallas import tpu as pltpu
from jax.experimental.pallas import tpu_sc as plsc

N = 131072
D = 128
H = 8
G = 1024
F = 1024
E = 16384
GG = G * G

def _gates_body(ea_t_ref, egs_ref, w2_ref, b2_ref, w1t_ref, b1t_ref,
                g_ref, cw_ref, cb_ref):
    z = jnp.dot(egs_ref[...], ea_t_ref[...], preferred_element_type=jnp.float32)
    g_ref[...] = jax.nn.sigmoid(z)
    lane_blk = lax.broadcasted_iota(jnp.int32, (1, D), 1) // H
    w1t = w1t_ref[...]
    for dd in range(16):
        b3dd = jnp.where(lane_blk == dd, w1t, 0.0)
        cw_ref[dd] = jnp.dot(w2_ref[...], b3dd,
                             preferred_element_type=jnp.float32)
    cb_ref[...] = jnp.dot(b2_ref[...], w1t,
                          preferred_element_type=jnp.float32) + b1t_ref[...]


def _gates_call(ea_t, egs, w2, b2, w1t, b1t):
    return pl.pallas_call(
        _gates_body,
        out_shape=(
            jax.ShapeDtypeStruct((2, E), jnp.float32),
            jax.ShapeDtypeStruct((16, D, D), jnp.float32),
            jax.ShapeDtypeStruct((1, D), jnp.float32),
        ),
    )(ea_t, egs, w2, b2, w1t, b1t)


_NSUB = 16
_EPW = E // _NSUB
_ZB = 16384
_OUTW = GG // _NSUB
_ROWS_PER_SUB = G // _NSUB


def _adj_body(ei_hbm, gates_hbm, out_hbm, src_v, dst_v, idx_v, val_v, zero_v,
              a_sh, sem):
    cid = lax.axis_index("c")
    sid = lax.axis_index("s")

    base = sid * _EPW
    pltpu.sync_copy(ei_hbm.at[0, pl.ds(base, _EPW)], src_v)
    pltpu.sync_copy(ei_hbm.at[1, pl.ds(base, _EPW)], dst_v)
    pltpu.sync_copy(gates_hbm.at[cid, pl.ds(base, _EPW)], val_v)

    def _zfill(i, _):
        zero_v[pl.ds(i * 16, 16)] = jnp.zeros((16,), jnp.float32)
        return 0
    lax.fori_loop(0, _ZB // 16, _zfill, 0)
    for j in range(_OUTW // _ZB):
        pltpu.sync_copy(zero_v, a_sh.at[pl.ds(sid * _OUTW + j * _ZB, _ZB)])

    def _idx(i, _):
        s16 = src_v[pl.ds(i * 16, 16)]
        d16 = dst_v[pl.ds(i * 16, 16)]
        idx_v[pl.ds(i * 16, 16)] = d16 * G + s16
        return 0
    lax.fori_loop(0, _EPW // 16, _idx, 0)

    plsc.subcore_barrier()

    pltpu.sync_copy(val_v, a_sh.at[idx_v], add=True)

    plsc.subcore_barrier()

    copies = []
    for r in range(_ROWS_PER_SUB):
        row = sid * _ROWS_PER_SUB + r
        copies.append(pltpu.make_async_copy(
            a_sh.at[pl.ds(row * G, G)],
            out_hbm.at[cid * G + row],
            sem,
        ))
    for cp in copies:
        cp.start()
    for cp in copies:
        cp.wait()


_adj_call = functools.partial(
    pl.kernel,
    mesh=plsc.VectorSubcoreMesh(core_axis_name="c", subcore_axis_name="s"),
    out_type=jax.ShapeDtypeStruct((2 * G, G), jnp.float32),
    scratch_types=[
        pltpu.VMEM((_EPW,), jnp.int32),
        pltpu.VMEM((_EPW,), jnp.int32),
        pltpu.VMEM((_EPW,), jnp.int32),
        pltpu.VMEM((_EPW,), jnp.float32),
        pltpu.VMEM((_ZB,), jnp.float32),
        pltpu.VMEM_SHARED((GG,), jnp.float32),
        pltpu.SemaphoreType.DMA,
    ],
)(_adj_body)


_BN = 16384
_BR = _BN // 16


def _fc_body(x_ref, cw_ref, cb_ref, o_ref):
    xw = x_ref[...].reshape(_BR, 16 * D)
    acc = jnp.dot(xw, cw_ref[...].reshape(16 * D, D),
                  preferred_element_type=jnp.float32)
    o_ref[...] = jnp.maximum(acc + cb_ref[...], 0.0)


def _fc_call(x, cw, cb):
    return pl.pallas_call(
        _fc_body,
        grid=(N // _BN,),
        in_specs=[
            pl.BlockSpec((_BN, D), lambda i: (i, 0)),
            pl.BlockSpec((16, D, D), lambda i: (0, 0, 0)),
            pl.BlockSpec((1, D), lambda i: (0, 0)),
        ],
        out_specs=pl.BlockSpec((_BR, D), lambda i: (i, 0)),
        out_shape=jax.ShapeDtypeStruct((N // 16, D), jnp.float32),
    )(x, cw, cb)


_HF = F // 2


def _convs_body(h2_ref, a_ref, wm0_ref, wr0_ref, b0_ref,
                wm1_ref, wr1_ref, b1_ref, o_ref, h1_s, wm1h_s, wr1h_s):
    i = pl.program_id(0)

    @pl.when(i == 0)
    def _conv0():
        h = h2_ref[...].reshape(G, F)
        hm = jnp.dot(h, wm0_ref[...], preferred_element_type=jnp.float32)
        hr = jnp.dot(h, wr0_ref[...], preferred_element_type=jnp.float32)
        agg = jnp.dot(a_ref[...], hm, preferred_element_type=jnp.float32)
        h1_s[...] = jnp.maximum(hr + agg + b0_ref[...], 0.0)
        wm1h_s[...] = wm1_ref[...]
        wr1h_s[...] = wr1_ref[...]

    @pl.when(i == 1)
    def _conv1():
        h1 = h1_s[...]
        h1a = h1[:, :_HF]
        h1b = h1[:, _HF:]
        hm1 = (jnp.dot(h1a, wm1h_s[...], preferred_element_type=jnp.float32) +
               jnp.dot(h1b, wm1_ref[...], preferred_element_type=jnp.float32))
        agg1 = jnp.dot(a_ref[...], hm1, preferred_element_type=jnp.float32)
        hr1 = (jnp.dot(h1a, wr1h_s[...], preferred_element_type=jnp.float32) +
               jnp.dot(h1b, wr1_ref[...], preferred_element_type=jnp.float32))
        o_ref[...] = jnp.maximum(hr1 + agg1 + b1_ref[...], 0.0)


def _convs_call(h2, adj, wm0, wr0, b0, wm1, wr1, b1):
    zero2 = lambda i: (0, 0)
    return pl.pallas_call(
        _convs_body,
        grid=(2,),
        in_specs=[
            pl.BlockSpec((N // 16, D), zero2),
            pl.BlockSpec((G, G), lambda i: (i, 0)),
            pl.BlockSpec((F, F), zero2),
            pl.BlockSpec((F, F), zero2),
            pl.BlockSpec((1, F), zero2),
            pl.BlockSpec((_HF, F), lambda i: (i, 0)),
            pl.BlockSpec((_HF, F), lambda i: (i, 0)),
            pl.BlockSpec((1, F), zero2),
        ],
        out_specs=pl.BlockSpec((G, F), zero2),
        out_shape=jax.ShapeDtypeStruct((G, F), jnp.float32),
        scratch_shapes=[
            pltpu.VMEM((G, F), jnp.float32),
            pltpu.VMEM((_HF, F), jnp.float32),
            pltpu.VMEM((_HF, F), jnp.float32),
        ],
    )(h2, adj, wm0, wr0, b0, wm1, wr1, b1)


def kernel(x, edge_index, edge_attr, fc2_w, fc2_b, fc1_w, fc1_b,
           conv0_root, conv0_msg, conv0_eg, conv0_b,
           conv1_root, conv1_msg, conv1_eg, conv1_b):
    ea_t = edge_attr.T
    egs = jnp.concatenate([conv0_eg.T, conv1_eg.T], axis=0)

    w1t = jnp.tile(fc1_w, (1, 16))
    b1t = jnp.tile(fc1_b, 16).reshape(1, D)

    gates, cw, cb = _gates_call(ea_t, egs, fc2_w, fc2_b.reshape(1, D),
                                w1t, b1t)

    adj = _adj_call(edge_index, gates)

    h2 = _fc_call(x, cw, cb)

    return _convs_call(h2, adj, conv0_msg, conv0_root, conv0_b.reshape(1, F),
                       conv1_msg, conv1_root, conv1_b.reshape(1, F))
